# baseline (speedup 1.0000x reference)
import functools

import jax
import jax.numpy as jnp
from jax import lax
from jax.experimental import pallas as pl
from jax.experimental.pallas import tpu as pltpu
from jax.experimental.pallas import tpu_sc as plsc

N_POINTS = 100000
N_NEIGH = 6
D_FEAT = 128

B = 125
G = N_POINTS // B
NW = 32
CPW = G // NW
RPW = B * CPW
LANES = 16
NS = 3
VPB = 8
BP = LANES * VPB


def _make_kernel():
    mesh = plsc.VectorSubcoreMesh(
        core_axis_name="c", subcore_axis_name="s",
        num_cores=2, num_subcores=16)

    scratch = (
        [pltpu.VMEM((RPW, N_NEIGH), jnp.int32)]
        + [pltpu.VMEM((NS, N_NEIGH, BP), jnp.int32)]
        + [pltpu.VMEM((BP, D_FEAT), jnp.float32) for _ in range(2 * NS)]
        + [pltpu.SemaphoreType.DMA for _ in range(2 * NS)]
    )

    @functools.partial(
        pl.kernel,
        out_type=jax.ShapeDtypeStruct((N_POINTS, D_FEAT), jnp.float32),
        mesh=mesh,
        scratch_types=scratch,
        compiler_params=pltpu.CompilerParams(use_tc_tiling_on_sc=False, needs_layout_passes=False),
    )
    def k(x_hbm, nbr_hbm, out_hbm, idx_raw, idx_dei,
          a0, a1, a2, b0, b1, b2, gs0, gs1, gs2, os0, os1, os2):
        wid = lax.axis_index("s") * 2 + lax.axis_index("c")
        base_g = wid * CPW
        a = [a0, a1, a2]
        bb = [b0, b1, b2]
        gs = [gs0, gs1, gs2]
        osem = [os0, os1, os2]
        inv = jnp.float32(1.0 / N_NEIGH)
        zeros = jnp.zeros((LANES,), jnp.float32)
        lanes = jnp.arange(LANES, dtype=jnp.int32)
        last_row = jnp.int32(RPW - 1)

        pltpu.sync_copy(nbr_hbm.at[pl.ds(wid * RPW, RPW)], idx_raw)

        def zero_buf(buf):
            def zb(r, c2):
                for c in range(D_FEAT // LANES):
                    buf[r, pl.ds(c * LANES, LANES)] = zeros
                return c2
            lax.fori_loop(0, B, zb, 0)

        for buf in bb:
            zero_buf(buf)

        def deinterleave(i):
            s = i % NS
            for v in range(VPB):
                rows = jnp.minimum(lanes + (i * B + v * LANES), last_row)
                for j in range(N_NEIGH):
                    col = jnp.full((LANES,), j, dtype=jnp.int32)
                    vals = plsc.load_gather(idx_raw, [rows, col])
                    idx_dei[s, j, pl.ds(v * LANES, LANES)] = vals

        def fire6(i):
            s = i % NS
            cps = [pltpu.async_copy(
                x_hbm.at[idx_dei.at[s, 0]], a[s], gs[s])]
            for j in range(1, N_NEIGH):
                cps.append(pltpu.async_copy(
                    x_hbm.at[idx_dei.at[s, j]], bb[s], gs[s],
                    add=True))
            return cps

        def compute(i):
            s = i % NS
            av, bv = a[s], bb[s]

            def row(r, c2):
                for c in range(D_FEAT // LANES):
                    sl = pl.ds(c * LANES, LANES)
                    av[r, sl] = (av[r, sl] + bv[r, sl]) * inv
                    bv[r, sl] = zeros
                return c2
            lax.fori_loop(0, B, row, 0)

        handles = [None] * CPW
        out_handles = [None] * CPW
        deinterleave(0)
        handles[0] = fire6(0)
        deinterleave(1)
        handles[1] = fire6(1)
        for i in range(CPW):
            if i + 2 < CPW:
                if i - 1 >= 0:
                    out_handles[i - 1].wait()
                deinterleave(i + 2)
                handles[i + 2] = fire6(i + 2)
            for cp in handles[i]:
                cp.wait()
            compute(i)
            s = i % NS
            out_handles[i] = pltpu.async_copy(
                a[s].at[pl.ds(0, B)], out_hbm.at[pl.ds((base_g + i) * B, B)], osem[s])
        for i in range(CPW - 3, CPW):
            out_handles[i].wait()

    return k


def kernel(x, neighbours):
    return _make_kernel()(x, neighbours)


# speedup vs baseline: 1.5589x; 1.5589x over previous
import functools

import jax
import jax.numpy as jnp
from jax import lax
from jax.experimental import pallas as pl
from jax.experimental.pallas import tpu as pltpu
from jax.experimental.pallas import tpu_sc as plsc

N_POINTS = 100000
N_NEIGH = 6
D_FEAT = 128

B = 125
G = N_POINTS // B
NW = 32
CPW = G // NW
LANES = 16
NS = 3


def _make_kernel():
    mesh = plsc.VectorSubcoreMesh(
        core_axis_name="c", subcore_axis_name="s",
        num_cores=2, num_subcores=16)

    scratch = (
        [pltpu.VMEM((CPW, N_NEIGH, B), jnp.int32)]
        + [pltpu.VMEM((B, D_FEAT), jnp.float32) for _ in range(2 * NS)]
        + [pltpu.SemaphoreType.DMA for _ in range(2 * NS)]
    )

    @functools.partial(
        pl.kernel,
        out_type=jax.ShapeDtypeStruct((N_POINTS, D_FEAT), jnp.float32),
        mesh=mesh,
        scratch_types=scratch,
        compiler_params=pltpu.CompilerParams(use_tc_tiling_on_sc=False),
    )
    def k(x_hbm, nbr_hbm, out_hbm, idx_all,
          a0, a1, a2, b0, b1, b2, gs0, gs1, gs2, os0, os1, os2):
        wid = lax.axis_index("s") * 2 + lax.axis_index("c")
        base_g = wid * CPW
        a = [a0, a1, a2]
        bb = [b0, b1, b2]
        gs = [gs0, gs1, gs2]
        osem = [os0, os1, os2]
        inv = jnp.float32(1.0 / N_NEIGH)
        zeros = jnp.zeros((LANES,), jnp.float32)

        pltpu.sync_copy(nbr_hbm.at[pl.ds(base_g, CPW)], idx_all)

        def zero_buf(buf):
            def zb(r, c2):
                for c in range(D_FEAT // LANES):
                    buf[r, pl.ds(c * LANES, LANES)] = zeros
                return c2
            lax.fori_loop(0, B, zb, 0)

        for buf in bb:
            zero_buf(buf)

        def fire6(i):
            s = i % NS
            cps = [pltpu.async_copy(x_hbm.at[idx_all.at[i, 0]], a[s], gs[s])]
            for j in range(1, N_NEIGH):
                cps.append(pltpu.async_copy(
                    x_hbm.at[idx_all.at[i, j]], bb[s], gs[s], add=True))
            return cps

        def compute(i):
            s = i % NS
            av, bv = a[s], bb[s]

            def row(r, c2):
                for c in range(D_FEAT // LANES):
                    sl = pl.ds(c * LANES, LANES)
                    av[r, sl] = (av[r, sl] + bv[r, sl]) * inv
                    bv[r, sl] = zeros
                return c2
            lax.fori_loop(0, B, row, 0)

        handles = [None] * CPW
        out_handles = [None] * CPW
        handles[0] = fire6(0)
        handles[1] = fire6(1)
        for i in range(CPW):
            if i + 2 < CPW:
                if i - 1 >= 0:
                    out_handles[i - 1].wait()
                handles[i + 2] = fire6(i + 2)
            for cp in handles[i]:
                cp.wait()
            compute(i)
            s = i % NS
            out_handles[i] = pltpu.async_copy(
                a[s], out_hbm.at[pl.ds((base_g + i) * B, B)], osem[s])
        for i in range(CPW - 3, CPW):
            out_handles[i].wait()

    return k


def kernel(x, neighbours):
    nbr3 = neighbours.reshape(G, B, N_NEIGH).transpose(0, 2, 1)
    return _make_kernel()(x, nbr3)
